# baseline (speedup 1.0000x reference)
import functools

import jax
import jax.numpy as jnp
from jax import lax
from jax.experimental import pallas as pl
from jax.experimental.pallas import tpu as pltpu
from jax.experimental.pallas import tpu_sc as plsc

_N_CAT = 26
_VOCAB = 100000
_EMBED = 32
_BATCH = 16384
_DENSE_DIM = 13
_NC = 2
_NS = 16
_NW = _NC * _NS
_BPW = _BATCH // _NW
_CH = 128
_NCHUNK = _BPW // _CH
_NT = _N_CAT * _NCHUNK


def _dense_body(x_ref, w_ref, b_ref, o_ref):
    acc = jnp.dot(x_ref[...], w_ref[...], preferred_element_type=jnp.float32)
    o_ref[...] = jnp.maximum(acc + b_ref[...], 0.0)


def _dense_tc(x, w, b2):
    blk = 2048
    return pl.pallas_call(
        _dense_body,
        grid=(_BATCH // blk,),
        in_specs=[
            pl.BlockSpec((blk, _DENSE_DIM), lambda i: (i, 0)),
            pl.BlockSpec((_DENSE_DIM, _EMBED), lambda i: (0, 0)),
            pl.BlockSpec((1, _EMBED), lambda i: (0, 0)),
        ],
        out_specs=pl.BlockSpec((blk, _EMBED), lambda i: (i, 0)),
        out_shape=jax.ShapeDtypeStruct((_BATCH, _EMBED), jnp.float32),
    )(x, w, b2)


@functools.partial(
    pl.kernel,
    mesh=plsc.VectorSubcoreMesh(core_axis_name="c", subcore_axis_name="s"),
    out_type=jax.ShapeDtypeStruct((_BATCH, _N_CAT + 1, _EMBED), jnp.float32),
    scratch_types=[
        pltpu.VMEM((_NT, _CH), jnp.int32),
        pltpu.VMEM((_CH, 1, _EMBED), jnp.float32),
        pltpu.VMEM((_BPW, 1, _EMBED), jnp.float32),
        pltpu.SemaphoreType.DMA,
    ],
    compiler_params=pltpu.CompilerParams(use_tc_tiling_on_sc=False),
)
def _sc_gather(idx_hbm, tab_hbm, dense_hbm, out_hbm, idx_v, rows_v, dense_v, sem):
    c = lax.axis_index("c")
    s = lax.axis_index("s")
    wid = s * _NC + c
    base = wid * _BPW
    pltpu.sync_copy(idx_hbm.at[wid], idx_v)

    def step(t, carry):
        feat = t // _NCHUNK
        boff = base + (t % _NCHUNK) * _CH
        pltpu.async_copy(tab_hbm.at[idx_v.at[t]], rows_v.at[:, 0], sem).wait()
        pltpu.sync_copy(rows_v, out_hbm.at[pl.ds(boff, _CH), pl.ds(feat, 1)])
        return carry

    lax.fori_loop(0, _NT, step, 0)
    pltpu.sync_copy(dense_hbm.at[pl.ds(base, _BPW)], dense_v.at[:, 0])
    pltpu.sync_copy(dense_v, out_hbm.at[pl.ds(base, _BPW), pl.ds(_N_CAT, 1)])


def kernel(cat_0, cat_1, cat_2, cat_3, cat_4, cat_5, cat_6, cat_7, cat_8,
           cat_9, cat_10, cat_11, cat_12, cat_13, cat_14, cat_15, cat_16,
           cat_17, cat_18, cat_19, cat_20, cat_21, cat_22, cat_23, cat_24,
           cat_25, dense_0, tables, W, b):
    cats = jnp.stack(
        [cat_0, cat_1, cat_2, cat_3, cat_4, cat_5, cat_6, cat_7, cat_8,
         cat_9, cat_10, cat_11, cat_12, cat_13, cat_14, cat_15, cat_16,
         cat_17, cat_18, cat_19, cat_20, cat_21, cat_22, cat_23, cat_24,
         cat_25], axis=0)
    offs = (jnp.arange(_N_CAT, dtype=jnp.int32) * _VOCAB)[:, None]
    idx = cats + offs
    idx = (idx.reshape(_N_CAT, _NW, _NCHUNK, _CH)
              .transpose(1, 0, 2, 3)
              .reshape(_NW, _NT, _CH))
    dense_out = _dense_tc(dense_0, W, b.reshape(1, _EMBED))
    tab = tables.reshape(_N_CAT * _VOCAB, _EMBED)
    return _sc_gather(idx, tab, dense_out)


# speedup vs baseline: 1.4400x; 1.4400x over previous
import functools

import jax
import jax.numpy as jnp
from jax import lax
from jax.experimental import pallas as pl
from jax.experimental.pallas import tpu as pltpu
from jax.experimental.pallas import tpu_sc as plsc

_N_CAT = 26
_VOCAB = 100000
_EMBED = 32
_BATCH = 16384
_DENSE_DIM = 13
_NC = 2
_NS = 16
_NW = _NC * _NS
_BPW = _BATCH // _NW
_CH = 128
_NCHUNK = _BPW // _CH
_NROW = _N_CAT * _EMBED


@functools.partial(
    pl.kernel,
    mesh=plsc.VectorSubcoreMesh(core_axis_name="c", subcore_axis_name="s"),
    out_type=jax.ShapeDtypeStruct((_N_CAT + 1, _EMBED, _BATCH), jnp.float32),
    scratch_types=[
        pltpu.VMEM((2, _BPW), jnp.int32),
        pltpu.VMEM((2, _EMBED, _BPW), jnp.float32),
        pltpu.VMEM((_DENSE_DIM, _BPW), jnp.float32),
        pltpu.VMEM((_DENSE_DIM * _EMBED,), jnp.float32),
        pltpu.VMEM((_EMBED,), jnp.float32),
        pltpu.SemaphoreType.DMA((2,)),
        pltpu.SemaphoreType.DMA((2,)),
    ],
    compiler_params=pltpu.CompilerParams(use_tc_tiling_on_sc=False),
)
def _sc_embed(idx_hbm, tab_hbm, dense_t_hbm, w_hbm, b_hbm, out_hbm,
              idx_v, slab_v, dt_v, w_v, b_v, gsem, ssem):
    c = lax.axis_index("c")
    s = lax.axis_index("s")
    wid = s * _NC + c
    base = wid * _BPW

    def fetch_idx(f, p):
        pltpu.sync_copy(idx_hbm.at[f, pl.ds(base, _BPW)], idx_v.at[p])

    def fire_feature(f, p):
        def per_dim(e, carry):
            row = f * _EMBED + e
            for ci in range(_NCHUNK):
                pltpu.async_copy(
                    tab_hbm.at[row].at[idx_v.at[p, pl.ds(ci * _CH, _CH)]],
                    slab_v.at[p, e, pl.ds(ci * _CH, _CH)],
                    gsem.at[p],
                )
            return carry

        lax.fori_loop(0, _EMBED, per_dim, 0)

    def drain(sem_arr, p):
        pltpu.make_async_copy(
            out_hbm.at[pl.ds(0, 1)],
            slab_v.at[pl.ds(p, 1)],
            sem_arr.at[p],
        ).wait()

    def scatter(f, p):
        pltpu.async_copy(
            slab_v.at[pl.ds(p, 1)],
            out_hbm.at[pl.ds(f, 1), :, pl.ds(base, _BPW)],
            ssem.at[p],
        )

    fetch_idx(0, 0)
    fire_feature(0, 0)

    def step(f, carry):
        p0 = lax.rem(f, 2)
        p1 = lax.rem(f + 1, 2)

        @pl.when(f + 1 < _N_CAT)
        def _():
            fetch_idx(f + 1, p1)

            @pl.when(f >= 1)
            def _():
                drain(ssem, p1)

            fire_feature(f + 1, p1)

        drain(gsem, p0)
        scatter(f, p0)
        return carry

    lax.fori_loop(0, _N_CAT, step, 0)

    pltpu.sync_copy(dense_t_hbm.at[:, pl.ds(base, _BPW)], dt_v)
    pltpu.sync_copy(w_hbm, w_v)
    pltpu.sync_copy(b_hbm, b_v)
    drain(ssem, 0)

    w_vecs = [w_v[pl.ds(i * 16, 16)] for i in range(_DENSE_DIM * _EMBED // 16)]
    b_vecs = [b_v[pl.ds(i * 16, 16)] for i in range(_EMBED // 16)]

    def per_group(g, carry):
        col = g * 16
        d = [dt_v[k, pl.ds(col, 16)] for k in range(_DENSE_DIM)]
        for e in range(_EMBED):
            acc = b_vecs[e // 16][e % 16] + jnp.zeros((16,), jnp.float32)
            for k in range(_DENSE_DIM):
                i = k * _EMBED + e
                acc = acc + d[k] * w_vecs[i // 16][i % 16]
            slab_v[0, e, pl.ds(col, 16)] = jnp.maximum(acc, 0.0)
        return carry

    lax.fori_loop(0, _BPW // 16, per_group, 0)
    scatter(_N_CAT, 0)
    drain(ssem, 1)
    drain(ssem, 0)


def kernel(cat_0, cat_1, cat_2, cat_3, cat_4, cat_5, cat_6, cat_7, cat_8,
           cat_9, cat_10, cat_11, cat_12, cat_13, cat_14, cat_15, cat_16,
           cat_17, cat_18, cat_19, cat_20, cat_21, cat_22, cat_23, cat_24,
           cat_25, dense_0, tables, W, b):
    idx = jnp.stack(
        [cat_0, cat_1, cat_2, cat_3, cat_4, cat_5, cat_6, cat_7, cat_8,
         cat_9, cat_10, cat_11, cat_12, cat_13, cat_14, cat_15, cat_16,
         cat_17, cat_18, cat_19, cat_20, cat_21, cat_22, cat_23, cat_24,
         cat_25], axis=0)
    tab_cols = tables.transpose(0, 2, 1).reshape(_NROW, _VOCAB)
    dense_t = dense_0.T
    out3 = _sc_embed(idx, tab_cols, dense_t, W.reshape(-1), b)
    return out3.transpose(2, 0, 1)


# speedup vs baseline: 1.5306x; 1.0629x over previous
import functools

import jax
import jax.numpy as jnp
from jax import lax
from jax.experimental import pallas as pl
from jax.experimental.pallas import tpu as pltpu
from jax.experimental.pallas import tpu_sc as plsc

_N_CAT = 26
_VOCAB = 100000
_EMBED = 32
_BATCH = 16384
_DENSE_DIM = 13
_NC = 2
_NS = 16
_NW = _NC * _NS
_BPW = _BATCH // _NW
_CH = 128
_NCHUNK = _BPW // _CH
_NROW = _N_CAT * _EMBED


@functools.partial(
    pl.kernel,
    mesh=plsc.VectorSubcoreMesh(core_axis_name="c", subcore_axis_name="s"),
    out_type=jax.ShapeDtypeStruct((_N_CAT + 1, 4, _BATCH // 128, 8, 128),
                                   jnp.float32),
    scratch_types=[
        pltpu.VMEM((2, _BPW), jnp.int32),
        pltpu.VMEM((2, 4, _NCHUNK, 8, 128), jnp.float32),
        pltpu.VMEM((_DENSE_DIM, _BPW), jnp.float32),
        pltpu.VMEM((_DENSE_DIM * _EMBED,), jnp.float32),
        pltpu.VMEM((_EMBED,), jnp.float32),
        pltpu.SemaphoreType.DMA((2,)),
        pltpu.SemaphoreType.DMA((2,)),
    ],
    compiler_params=pltpu.CompilerParams(use_tc_tiling_on_sc=False),
)
def _sc_embed(idx_hbm, tab_hbm, dense_t_hbm, w_hbm, b_hbm, out_hbm,
              idx_v, slab_v, dt_v, w_v, b_v, gsem, ssem):
    c = lax.axis_index("c")
    s = lax.axis_index("s")
    wid = s * _NC + c
    base = wid * _BPW

    def fetch_idx(f, p):
        pltpu.sync_copy(idx_hbm.at[f, pl.ds(base, _BPW)], idx_v.at[p])

    def fire_feature(f, p):
        def per_dim(e, carry):
            row = f * _EMBED + e
            for ci in range(_NCHUNK):
                pltpu.async_copy(
                    tab_hbm.at[row].at[idx_v.at[p, pl.ds(ci * _CH, _CH)]],
                    slab_v.at[p, e // 8, ci, lax.rem(e, 8), :],
                    gsem.at[p],
                )
            return carry

        lax.fori_loop(0, _EMBED, per_dim, 0)

    def drain(sem_arr, p):
        pltpu.make_async_copy(
            out_hbm.at[pl.ds(0, 1), :, pl.ds(0, _NCHUNK)],
            slab_v.at[pl.ds(p, 1)],
            sem_arr.at[p],
        ).wait()

    def scatter(f, p):
        pltpu.async_copy(
            slab_v.at[pl.ds(p, 1)],
            out_hbm.at[pl.ds(f, 1), :, pl.ds(wid * _NCHUNK, _NCHUNK)],
            ssem.at[p],
        )

    fetch_idx(0, 0)
    fire_feature(0, 0)

    def step(f, carry):
        p0 = lax.rem(f, 2)
        p1 = lax.rem(f + 1, 2)

        @pl.when(f + 1 < _N_CAT)
        def _():
            fetch_idx(f + 1, p1)

            @pl.when(f >= 1)
            def _():
                drain(ssem, p1)

            fire_feature(f + 1, p1)

        drain(gsem, p0)
        scatter(f, p0)
        return carry

    lax.fori_loop(0, _N_CAT, step, 0)

    pltpu.sync_copy(dense_t_hbm.at[:, pl.ds(base, _BPW)], dt_v)
    pltpu.sync_copy(w_hbm, w_v)
    pltpu.sync_copy(b_hbm, b_v)
    drain(ssem, 0)

    w_vecs = [w_v[pl.ds(i * 16, 16)] for i in range(_DENSE_DIM * _EMBED // 16)]
    b_vecs = [b_v[pl.ds(i * 16, 16)] for i in range(_EMBED // 16)]

    def per_group(g, carry):
        col = g * 16
        d = [dt_v[k, pl.ds(col, 16)] for k in range(_DENSE_DIM)]
        for e in range(_EMBED):
            acc = b_vecs[e // 16][e % 16] + jnp.zeros((16,), jnp.float32)
            for k in range(_DENSE_DIM):
                i = k * _EMBED + e
                acc = acc + d[k] * w_vecs[i // 16][i % 16]
            slab_v[0, e // 8, g // 8, lax.rem(e, 8),
                   pl.ds(lax.rem(g, 8) * 16, 16)] = jnp.maximum(acc, 0.0)
        return carry

    lax.fori_loop(0, _BPW // 16, per_group, 0)
    scatter(_N_CAT, 0)
    drain(ssem, 1)
    drain(ssem, 0)


def kernel(cat_0, cat_1, cat_2, cat_3, cat_4, cat_5, cat_6, cat_7, cat_8,
           cat_9, cat_10, cat_11, cat_12, cat_13, cat_14, cat_15, cat_16,
           cat_17, cat_18, cat_19, cat_20, cat_21, cat_22, cat_23, cat_24,
           cat_25, dense_0, tables, W, b):
    idx = jnp.stack(
        [cat_0, cat_1, cat_2, cat_3, cat_4, cat_5, cat_6, cat_7, cat_8,
         cat_9, cat_10, cat_11, cat_12, cat_13, cat_14, cat_15, cat_16,
         cat_17, cat_18, cat_19, cat_20, cat_21, cat_22, cat_23, cat_24,
         cat_25], axis=0)
    tab_cols = tables.transpose(0, 2, 1).reshape(_NROW, _VOCAB)
    dense_t = dense_0.T
    out5 = _sc_embed(idx, tab_cols, dense_t, W.reshape(-1), b)
    return out5.transpose(2, 4, 0, 1, 3).reshape(_BATCH, _N_CAT + 1, _EMBED)
